# baseline (speedup 1.0000x reference)
import jax
import jax.numpy as jnp
from jax.experimental import pallas as pl
from jax.experimental.pallas import tpu as pltpu
from jax.experimental.pallas import tpu_sc as plsc


GATHER_WINDOW = 128


def _sc_gather(table, indices):
    b = indices.shape[1]
    d = table.shape[1]
    mesh = plsc.VectorSubcoreMesh(core_axis_name="core", subcore_axis_name="subcore")

    @pl.kernel(
        out_type=jax.ShapeDtypeStruct((b, d), table.dtype),
        mesh=mesh,
        scratch_types=[pltpu.SemaphoreType.DMA],
    )
    def kern(x_hbm, i_hbm, o_hbm, sem):
        def body(i_vmem, o_vmem):
            @pl.loop(0, GATHER_WINDOW)
            def _issue(j):
                row = i_vmem[0, pl.ds(j, 1)][0]
                pltpu.make_async_copy(x_hbm.at[row], o_vmem.at[j], sem).start()

            @pl.loop(0, GATHER_WINDOW)
            def _wait(j):
                row = i_vmem[0, pl.ds(j, 1)][0]
                pltpu.make_async_copy(x_hbm.at[row], o_vmem.at[j], sem).wait()

        pltpu.emit_pipeline(
            body,
            grid=(b // GATHER_WINDOW,),
            in_specs=[pl.BlockSpec((1, GATHER_WINDOW), index_map=lambda i: (0, i))],
            out_specs=[pl.BlockSpec((GATHER_WINDOW, d), index_map=lambda i: (i, 0))],
            core_axis_name=("core", "subcore"),
            dimension_semantics=(pltpu.PARALLEL,),
        )(i_hbm, o_hbm)

    return kern(table, indices)


def _expand_body(emb_ref, ft_ref, o_ref):
    emb = emb_ref[...]
    ft = ft_ref[...]
    bb, t, _ = ft.shape
    rep = jnp.broadcast_to(emb[:, None, :], (bb, t, emb.shape[1]))
    o_ref[...] = jnp.concatenate([rep, ft], axis=-1)


def _tc_expand(emb, ft, block_b=256):
    b, t, f = ft.shape
    d = emb.shape[1]
    return pl.pallas_call(
        _expand_body,
        grid=(b // block_b,),
        in_specs=[
            pl.BlockSpec((block_b, d), lambda i: (i, 0)),
            pl.BlockSpec((block_b, t, f), lambda i: (i, 0, 0)),
        ],
        out_specs=pl.BlockSpec((block_b, t, d + f), lambda i: (i, 0, 0)),
        out_shape=jax.ShapeDtypeStruct((b, t, d + f), jnp.float32),
    )(emb, ft)


def kernel(feat_static, n_timesteps, feat_time, embedding_weight):
    idx = jnp.squeeze(feat_static.astype(jnp.int32), axis=-1).reshape(1, -1)
    emb = _sc_gather(embedding_weight, idx)
    return _tc_expand(emb, feat_time)
